# baseline (speedup 1.0000x reference)
import jax
import jax.numpy as jnp
import numpy as np
from jax import lax
from jax.experimental import pallas as pl
from jax.experimental.pallas import tpu as pltpu
from jax.experimental.pallas import tpu_sc as plsc

DIM = 4096
NLEV = 21
NCHUNK = 8
LANES = 16
NWORD = NCHUNK * LANES
XLEN = 600
NJ = 598
NJ_GROUPS = 74
NJ_REM = 6
NCLS = 5
THRESH = 299


def _mk_pack_m():
    m = np.zeros((DIM, 2 * (DIM // 32)), np.float32)
    w = np.arange(DIM) // 32
    b = np.arange(DIM) % 32
    m[np.arange(DIM), 2 * w + (b >= 16)] = 2.0 ** (b % 16)
    return m


_PACK_M = _mk_pack_m()


def _csa(a, b, cin):
    u = a ^ b
    return u ^ cin, (a & b) | (u & cin)


def _tree8(x, ones, twos, fours):
    s0, c0 = _csa(x[0], x[1], x[2])
    s1, c1 = _csa(x[3], x[4], x[5])
    s2, c2 = _csa(x[6], x[7], s0)
    ones, c3 = _csa(s1, s2, ones)
    t0, d0 = _csa(c0, c1, c2)
    twos, d1 = _csa(c3, t0, twos)
    fours, e0 = _csa(d0, d1, fours)
    return ones, twos, fours, e0


def _sc_body(xin, p0pad, wp, out, xin_v, p0pad_v, ptab_v, wp_v, out_v, offs_s,
             sem):
    wid = lax.axis_index("s") * 2 + lax.axis_index("c")
    wp_copy = pltpu.async_copy(wp, wp_v, sem)
    pltpu.sync_copy(xin.at[wid], xin_v)
    pltpu.sync_copy(p0pad, p0pad_v)

    def quant(base):
        v = xin_v[pl.ds(base, LANES)]
        u = (v / 20.0) * 20.0
        h = u + 0.5
        r = h.astype(jnp.int32)
        is_half = r.astype(jnp.float32) == h
        r = r - jnp.where(is_half, r & 1, 0)
        r = jnp.minimum(jnp.maximum(r, 0), NLEV - 1)
        roff = r * NWORD
        for l in range(LANES):
            offs_s[base + l] = roff[l]

    for t in range(XLEN // LANES):
        quant(t * LANES)
    quant(XLEN - LANES)

    def mkrow(i, carry):
        for c in range(NCHUNK):
            a = p0pad_v[pl.ds(i * (NWORD + 1) + 1 + c * LANES, LANES)]
            ap = p0pad_v[pl.ds(i * (NWORD + 1) + c * LANES, LANES)]
            base = i * NWORD + c * LANES
            ptab_v[pl.ds(base, LANES)] = (
                (a << 2) | lax.shift_right_logical(ap, 30))
            ptab_v[pl.ds(NLEV * NWORD + base, LANES)] = (
                (a << 1) | lax.shift_right_logical(ap, 31))
            ptab_v[pl.ds(2 * NLEV * NWORD + base, LANES)] = a
        return carry

    lax.fori_loop(0, NLEV, mkrow, 0)

    def product(ovals, k, c):
        w2 = ptab_v[pl.ds(ovals[k] + c * LANES, LANES)]
        w1 = ptab_v[pl.ds(ovals[k + 1] + (NLEV * NWORD + c * LANES), LANES)]
        w0 = ptab_v[pl.ds(ovals[k + 2] + (2 * NLEV * NWORD + c * LANES), LANES)]
        return (w2 ^ w1) ^ w0

    wp_copy.wait()

    acc = [jnp.zeros((LANES,), jnp.float32) for _ in range(NCLS)]
    zero = jnp.zeros((LANES,), jnp.int32)

    for c in range(NCHUNK):
        def group(g, st, c=c):
            ones, twos, fours, p3, p4, p5, p6, p7, p8, p9 = st
            j0 = g * 8
            ovals = [offs_s[j0 + i] for i in range(10)]
            x = [product(ovals, k, c) for k in range(8)]
            ones, twos, fours, carry = _tree8(x, ones, twos, fours)
            ps = [p3, p4, p5, p6, p7, p8, p9]
            for i in range(7):
                nxt = ps[i] ^ carry
                carry = ps[i] & carry
                ps[i] = nxt
            return (ones, twos, fours, *ps)

        planes = list(lax.fori_loop(0, NJ_GROUPS, group, (zero,) * 10))

        orem = [offs_s[NJ_GROUPS * 8 + i] for i in range(NJ_REM + 2)]
        for k in range(NJ_REM):
            carry = product(orem, k, c)
            for i in range(10):
                nxt = planes[i] ^ carry
                carry = planes[i] & carry
                planes[i] = nxt

        kadd = 1024 - THRESH
        carry = zero
        for p in range(10):
            if (kadd >> p) & 1:
                carry = planes[p] | carry
            else:
                carry = planes[p] & carry
        sbits = carry

        def clsbody(q, accs, c=c, sbits=sbits):
            accs = list(accs)
            for u in range(4):
                bit = q * 4 + u
                sgnbit = jnp.left_shift(jnp.right_shift(sbits, bit) & 1, 31)
                base = (c * 32) * LANES + bit * LANES
                accs = [
                    a + lax.bitcast_convert_type(
                        lax.bitcast_convert_type(
                            wp_v[pl.ds(k * (NCHUNK * 32 * LANES) + base,
                                       LANES)],
                            jnp.int32) ^ sgnbit,
                        jnp.float32)
                    for k, a in enumerate(accs)]
            return tuple(accs)

        acc = list(lax.fori_loop(0, 8, clsbody, tuple(acc)))

    io = lax.broadcasted_iota(jnp.int32, (LANES,), 0)
    ov = jnp.zeros((LANES,), jnp.float32)
    for k in range(NCLS):
        total = acc[k][0]
        for l in range(1, LANES):
            total = total + acc[k][l]
        ov = jnp.where(io == k, total, ov)
    out_v[...] = ov
    pltpu.sync_copy(out_v, out.at[wid])


def kernel(x, level_hv, W):
    batch = x.shape[0]
    xin = x.reshape(batch, -1)

    m = level_hv @ jnp.asarray(_PACK_M)
    halves = ((65535.0 - m) * 0.5).astype(jnp.int32).reshape(NLEV, NWORD, 2)
    p0 = halves[..., 0] | (halves[..., 1] << 16)
    p0pad = jnp.concatenate([p0[:, -1:], p0], axis=1).reshape(-1)

    wp = W.reshape(NCLS, NCHUNK, LANES, 32).transpose(0, 1, 3, 2)

    mesh = plsc.VectorSubcoreMesh(core_axis_name="c", subcore_axis_name="s")
    out = pl.kernel(
        _sc_body,
        mesh=mesh,
        out_type=jax.ShapeDtypeStruct((batch, LANES), jnp.float32),
        scratch_types=[
            pltpu.VMEM((XLEN,), jnp.float32),
            pltpu.VMEM((NLEV * (NWORD + 1),), jnp.int32),
            pltpu.VMEM((3 * NLEV * NWORD,), jnp.int32),
            pltpu.VMEM((NCLS * NCHUNK * 32 * LANES,), jnp.float32),
            pltpu.VMEM((LANES,), jnp.float32),
            pltpu.SMEM((XLEN,), jnp.int32),
            pltpu.SemaphoreType.DMA,
        ],
    )(xin, p0pad, wp.reshape(-1))
    return out[:, :NCLS]


# speedup vs baseline: 1.0185x; 1.0185x over previous
import jax
import jax.numpy as jnp
from jax import lax
from jax.experimental import pallas as pl
from jax.experimental.pallas import tpu as pltpu
from jax.experimental.pallas import tpu_sc as plsc

DIM = 4096
NLEV = 21
NCHUNK = 8
LANES = 16
NWORD = NCHUNK * LANES
XLEN = 600
NJ = 598
NJ_GROUPS = 74
NJ_REM = 6
NCLS = 5
THRESH = 299


def _csa(a, b, cin):
    u = a ^ b
    return u ^ cin, (a & b) | (u & cin)


def _tree8(x, ones, twos, fours):
    s0, c0 = _csa(x[0], x[1], x[2])
    s1, c1 = _csa(x[3], x[4], x[5])
    s2, c2 = _csa(x[6], x[7], s0)
    ones, c3 = _csa(s1, s2, ones)
    t0, d0 = _csa(c0, c1, c2)
    twos, d1 = _csa(c3, t0, twos)
    fours, e0 = _csa(d0, d1, fours)
    return ones, twos, fours, e0


def _sc_body(xin, p0pad, wp, out, xin_v, p0pad_v, ptab_v, wp_v, out_v, offs_s,
             sem):
    wid = lax.axis_index("s") * 2 + lax.axis_index("c")
    wp_copy = pltpu.async_copy(wp, wp_v, sem)
    pltpu.sync_copy(xin.at[wid], xin_v)
    pltpu.sync_copy(p0pad, p0pad_v)

    def quant(base):
        v = xin_v[pl.ds(base, LANES)]
        u = (v / 20.0) * 20.0
        h = u + 0.5
        r = h.astype(jnp.int32)
        is_half = r.astype(jnp.float32) == h
        r = r - jnp.where(is_half, r & 1, 0)
        r = jnp.minimum(jnp.maximum(r, 0), NLEV - 1)
        roff = r * NWORD
        for l in range(LANES):
            offs_s[base + l] = roff[l]

    for t in range(XLEN // LANES):
        quant(t * LANES)
    quant(XLEN - LANES)

    def mkrow(i, carry):
        for c in range(NCHUNK):
            a = p0pad_v[pl.ds(i * (NWORD + 1) + 1 + c * LANES, LANES)]
            ap = p0pad_v[pl.ds(i * (NWORD + 1) + c * LANES, LANES)]
            base = i * NWORD + c * LANES
            ptab_v[pl.ds(base, LANES)] = (
                (a << 2) | lax.shift_right_logical(ap, 30))
            ptab_v[pl.ds(NLEV * NWORD + base, LANES)] = (
                (a << 1) | lax.shift_right_logical(ap, 31))
            ptab_v[pl.ds(2 * NLEV * NWORD + base, LANES)] = a
        return carry

    lax.fori_loop(0, NLEV, mkrow, 0)

    def product(ovals, k, c):
        w2 = ptab_v[pl.ds(ovals[k] + c * LANES, LANES)]
        w1 = ptab_v[pl.ds(ovals[k + 1] + (NLEV * NWORD + c * LANES), LANES)]
        w0 = ptab_v[pl.ds(ovals[k + 2] + (2 * NLEV * NWORD + c * LANES), LANES)]
        return (w2 ^ w1) ^ w0

    wp_copy.wait()

    acc = [jnp.zeros((LANES,), jnp.float32) for _ in range(NCLS)]
    zero = jnp.zeros((LANES,), jnp.int32)

    for c in range(NCHUNK):
        def group(g, st, c=c):
            ones, twos, fours, p3, p4, p5, p6, p7, p8, p9 = st
            j0 = g * 8
            ovals = [offs_s[j0 + i] for i in range(10)]
            x = [product(ovals, k, c) for k in range(8)]
            ones, twos, fours, carry = _tree8(x, ones, twos, fours)
            ps = [p3, p4, p5, p6, p7, p8, p9]
            for i in range(7):
                nxt = ps[i] ^ carry
                carry = ps[i] & carry
                ps[i] = nxt
            return (ones, twos, fours, *ps)

        planes = list(lax.fori_loop(0, NJ_GROUPS, group, (zero,) * 10))

        orem = [offs_s[NJ_GROUPS * 8 + i] for i in range(NJ_REM + 2)]
        for k in range(NJ_REM):
            carry = product(orem, k, c)
            for i in range(10):
                nxt = planes[i] ^ carry
                carry = planes[i] & carry
                planes[i] = nxt

        kadd = 1024 - THRESH
        carry = zero
        for p in range(10):
            if (kadd >> p) & 1:
                carry = planes[p] | carry
            else:
                carry = planes[p] & carry
        sbits = carry

        def clsbody(q, accs, c=c, sbits=sbits):
            accs = list(accs)
            for u in range(4):
                bit = q * 4 + u
                sgnbit = jnp.left_shift(jnp.right_shift(sbits, bit) & 1, 31)
                base = (c * 32) * LANES + bit * LANES
                accs = [
                    a + lax.bitcast_convert_type(
                        lax.bitcast_convert_type(
                            wp_v[pl.ds(k * (NCHUNK * 32 * LANES) + base,
                                       LANES)],
                            jnp.int32) ^ sgnbit,
                        jnp.float32)
                    for k, a in enumerate(accs)]
            return tuple(accs)

        acc = list(lax.fori_loop(0, 8, clsbody, tuple(acc)))

    io = lax.broadcasted_iota(jnp.int32, (LANES,), 0)
    ov = jnp.zeros((LANES,), jnp.float32)
    for k in range(NCLS):
        total = acc[k][0]
        for l in range(1, LANES):
            total = total + acc[k][l]
        ov = jnp.where(io == k, total, ov)
    out_v[...] = ov
    pltpu.sync_copy(out_v, out.at[wid])


def kernel(x, level_hv, W):
    batch = x.shape[0]
    xin = x.reshape(batch, -1)

    pows = jnp.zeros((32, 2), jnp.float32)
    pows = pows.at[:16, 0].set(2.0 ** jnp.arange(16, dtype=jnp.float32))
    pows = pows.at[16:, 1].set(2.0 ** jnp.arange(16, dtype=jnp.float32))
    m = level_hv.reshape(NLEV * NWORD, 32) @ pows
    halves = ((65535.0 - m) * 0.5).astype(jnp.int32)
    p0 = (halves[:, 0] | (halves[:, 1] << 16)).reshape(NLEV, NWORD)
    p0pad = jnp.concatenate([p0[:, -1:], p0], axis=1).reshape(-1)

    wp = W.reshape(NCLS, NCHUNK, LANES, 32).transpose(0, 1, 3, 2)

    mesh = plsc.VectorSubcoreMesh(core_axis_name="c", subcore_axis_name="s")
    out = pl.kernel(
        _sc_body,
        mesh=mesh,
        out_type=jax.ShapeDtypeStruct((batch, LANES), jnp.float32),
        scratch_types=[
            pltpu.VMEM((XLEN,), jnp.float32),
            pltpu.VMEM((NLEV * (NWORD + 1),), jnp.int32),
            pltpu.VMEM((3 * NLEV * NWORD,), jnp.int32),
            pltpu.VMEM((NCLS * NCHUNK * 32 * LANES,), jnp.float32),
            pltpu.VMEM((LANES,), jnp.float32),
            pltpu.SMEM((XLEN,), jnp.int32),
            pltpu.SemaphoreType.DMA,
        ],
    )(xin, p0pad, wp.reshape(-1))
    return out[:, :NCLS]


# speedup vs baseline: 1.0370x; 1.0182x over previous
import jax
import jax.numpy as jnp
from jax import lax
from jax.experimental import pallas as pl
from jax.experimental.pallas import tpu as pltpu
from jax.experimental.pallas import tpu_sc as plsc

DIM = 4096
NLEV = 21
NCHUNK = 8
LANES = 16
NWORD = NCHUNK * LANES
XLEN = 600
NJ = 598
NJ_GROUPS = 74
NJ_REM = 6
NCLS = 5
THRESH = 299


def _csa(a, b, cin):
    u = a ^ b
    return u ^ cin, (a & b) | (u & cin)


def _tree8(x, ones, twos, fours):
    s0, c0 = _csa(x[0], x[1], x[2])
    s1, c1 = _csa(x[3], x[4], x[5])
    s2, c2 = _csa(x[6], x[7], s0)
    ones, c3 = _csa(s1, s2, ones)
    t0, d0 = _csa(c0, c1, c2)
    twos, d1 = _csa(c3, t0, twos)
    fours, e0 = _csa(d0, d1, fours)
    return ones, twos, fours, e0


def _sc_body(xin, p0pad, wp, out, xin_v, p0pad_v, ptab_v, wp_v, out_v, offs_s,
             sem):
    wid = lax.axis_index("s") * 2 + lax.axis_index("c")
    wp_copy = pltpu.async_copy(wp, wp_v, sem)
    pltpu.sync_copy(xin.at[wid], xin_v)
    pltpu.sync_copy(p0pad, p0pad_v)

    def quant(base):
        v = xin_v[pl.ds(base, LANES)]
        u = (v / 20.0) * 20.0
        h = u + 0.5
        r = h.astype(jnp.int32)
        is_half = r.astype(jnp.float32) == h
        r = r - jnp.where(is_half, r & 1, 0)
        r = jnp.minimum(jnp.maximum(r, 0), NLEV - 1)
        roff = r * NWORD
        for l in range(LANES):
            offs_s[base + l] = roff[l]

    for t in range(XLEN // LANES):
        quant(t * LANES)
    quant(XLEN - LANES)

    def mkrow(i):
        for c in range(NCHUNK):
            a = p0pad_v[pl.ds(i * (NWORD + 1) + 1 + c * LANES, LANES)]
            ap = p0pad_v[pl.ds(i * (NWORD + 1) + c * LANES, LANES)]
            base = i * NWORD + c * LANES
            ptab_v[pl.ds(base, LANES)] = (
                (a << 2) | lax.shift_right_logical(ap, 30))
            ptab_v[pl.ds(NLEV * NWORD + base, LANES)] = (
                (a << 1) | lax.shift_right_logical(ap, 31))
            ptab_v[pl.ds(2 * NLEV * NWORD + base, LANES)] = a

    plsc.parallel_loop(0, NLEV)(mkrow)

    def product(ovals, k, c):
        w2 = ptab_v[pl.ds(ovals[k] + c * LANES, LANES)]
        w1 = ptab_v[pl.ds(ovals[k + 1] + (NLEV * NWORD + c * LANES), LANES)]
        w0 = ptab_v[pl.ds(ovals[k + 2] + (2 * NLEV * NWORD + c * LANES), LANES)]
        return (w2 ^ w1) ^ w0

    wp_copy.wait()

    acc = [jnp.zeros((LANES,), jnp.float32) for _ in range(NCLS)]
    zero = jnp.zeros((LANES,), jnp.int32)

    for c in range(NCHUNK):
        def group(g, st, c=c):
            ones, twos, fours, p3, p4, p5, p6, p7, p8, p9 = st
            j0 = g * 8
            ovals = [offs_s[j0 + i] for i in range(10)]
            x = [product(ovals, k, c) for k in range(8)]
            ones, twos, fours, carry = _tree8(x, ones, twos, fours)
            ps = [p3, p4, p5, p6, p7, p8, p9]
            for i in range(7):
                nxt = ps[i] ^ carry
                carry = ps[i] & carry
                ps[i] = nxt
            return (ones, twos, fours, *ps)

        planes = list(plsc.parallel_loop(0, NJ_GROUPS, carry=(zero,) * 10)(group))

        orem = [offs_s[NJ_GROUPS * 8 + i] for i in range(NJ_REM + 2)]
        for k in range(NJ_REM):
            carry = product(orem, k, c)
            for i in range(10):
                nxt = planes[i] ^ carry
                carry = planes[i] & carry
                planes[i] = nxt

        kadd = 1024 - THRESH
        carry = zero
        for p in range(10):
            if (kadd >> p) & 1:
                carry = planes[p] | carry
            else:
                carry = planes[p] & carry
        sbits = carry

        def clsbody(q, accs, c=c, sbits=sbits):
            accs = list(accs)
            for u in range(4):
                bit = q * 4 + u
                sgnbit = jnp.left_shift(jnp.right_shift(sbits, bit) & 1, 31)
                base = (c * 32) * LANES + bit * LANES
                accs = [
                    a + lax.bitcast_convert_type(
                        lax.bitcast_convert_type(
                            wp_v[pl.ds(k * (NCHUNK * 32 * LANES) + base,
                                       LANES)],
                            jnp.int32) ^ sgnbit,
                        jnp.float32)
                    for k, a in enumerate(accs)]
            return tuple(accs)

        acc = list(lax.fori_loop(0, 8, clsbody, tuple(acc)))

    io = lax.broadcasted_iota(jnp.int32, (LANES,), 0)
    ov = jnp.zeros((LANES,), jnp.float32)
    for k in range(NCLS):
        total = acc[k][0]
        for l in range(1, LANES):
            total = total + acc[k][l]
        ov = jnp.where(io == k, total, ov)
    out_v[...] = ov
    pltpu.sync_copy(out_v, out.at[wid])


def kernel(x, level_hv, W):
    batch = x.shape[0]
    xin = x.reshape(batch, -1)

    pows = jnp.zeros((32, 2), jnp.float32)
    pows = pows.at[:16, 0].set(2.0 ** jnp.arange(16, dtype=jnp.float32))
    pows = pows.at[16:, 1].set(2.0 ** jnp.arange(16, dtype=jnp.float32))
    m = level_hv.reshape(NLEV * NWORD, 32) @ pows
    halves = ((65535.0 - m) * 0.5).astype(jnp.int32)
    p0 = (halves[:, 0] | (halves[:, 1] << 16)).reshape(NLEV, NWORD)
    p0pad = jnp.concatenate([p0[:, -1:], p0], axis=1).reshape(-1)

    wp = W.reshape(NCLS, NCHUNK, LANES, 32).transpose(0, 1, 3, 2)

    mesh = plsc.VectorSubcoreMesh(core_axis_name="c", subcore_axis_name="s")
    out = pl.kernel(
        _sc_body,
        mesh=mesh,
        out_type=jax.ShapeDtypeStruct((batch, LANES), jnp.float32),
        scratch_types=[
            pltpu.VMEM((XLEN,), jnp.float32),
            pltpu.VMEM((NLEV * (NWORD + 1),), jnp.int32),
            pltpu.VMEM((3 * NLEV * NWORD,), jnp.int32),
            pltpu.VMEM((NCLS * NCHUNK * 32 * LANES,), jnp.float32),
            pltpu.VMEM((LANES,), jnp.float32),
            pltpu.SMEM((XLEN,), jnp.int32),
            pltpu.SemaphoreType.DMA,
        ],
    )(xin, p0pad, wp.reshape(-1))
    return out[:, :NCLS]
